# baseline (speedup 1.0000x reference)
import functools

import jax
import jax.numpy as jnp
from jax import lax
from jax.experimental import pallas as pl
from jax.experimental.pallas import tpu as pltpu
from jax.experimental.pallas import tpu_sc as plsc

_BATCH = 16384
_DIM = 128
_NUM_CORES = 2
_NUM_SUBCORES = 16
_NW = _NUM_CORES * _NUM_SUBCORES
_B_SC = 14336
_CHUNK = _B_SC // _NW
_S = 56
_NSUB = _CHUNK // _S
_LANES = 16

_mesh = plsc.VectorSubcoreMesh(core_axis_name="c", subcore_axis_name="s")


@functools.partial(
    pl.kernel,
    out_type=jax.ShapeDtypeStruct((_NW, _LANES), jnp.float32),
    mesh=_mesh,
    scratch_types=[
        pltpu.VMEM((_CHUNK,), jnp.int32),
        pltpu.VMEM((3, _S, _DIM), jnp.float32),
        pltpu.VMEM((_LANES,), jnp.float32),
        [pltpu.SemaphoreType.DMA] * 3,
        [pltpu.SemaphoreType.DMA] * 3,
    ],
)
def _center_loss_partials(x_hbm, labels_hbm, neg_centers_hbm, out_hbm,
                          idx_v, xc_v, acc_v, sems_x, sems_a):
    wid = lax.axis_index("s") * _NUM_CORES + lax.axis_index("c")
    base = wid * _CHUNK
    pltpu.sync_copy(labels_hbm.at[pl.ds(base, _CHUNK)], idx_v)

    def start_x(h):
        b = h % 3
        return pltpu.async_copy(
            x_hbm.at[pl.ds(base + h * _S, _S)], xc_v.at[b], sems_x[b])

    def start_add(h):
        b = h % 3
        return pltpu.async_copy(
            neg_centers_hbm.at[idx_v.at[pl.ds(h * _S, _S)]], xc_v.at[b],
            sems_a[b], add=True)

    cp_x = [None, None, None]
    cp_a = [None, None, None]
    cp_x[0] = start_x(0)
    cp_x[1] = start_x(1)
    cp_x[0].wait()
    cp_a[0] = start_add(0)
    cp_x[2] = start_x(2)

    acc = jnp.zeros((_LANES,), jnp.float32)
    for h in range(_NSUB):
        b = h % 3
        cp_a[b].wait()
        if h + 1 < _NSUB:
            cp_x[(h + 1) % 3].wait()
            cp_a[(h + 1) % 3] = start_add(h + 1)

        def row_body(r, a):
            for j in range(_DIM // _LANES):
                d = xc_v[b, r, pl.ds(j * _LANES, _LANES)]
                a = a + d * d
            return a

        acc = lax.fori_loop(0, _S, row_body, acc)
        if h + 3 < _NSUB:
            cp_x[(h + 3) % 3] = start_x(h + 3)

    acc_v[...] = acc
    pltpu.sync_copy(acc_v, out_hbm.at[wid])


def kernel(x, labels, centers):
    partials = _center_loss_partials(x, labels, -centers)
    c_tc = jnp.take(centers, labels[_B_SC:], axis=0)
    tc_part = jnp.sum(jnp.square(x[_B_SC:] - c_tc))
    return (jnp.sum(partials) + tc_part) * (0.5 / _BATCH)


# speedup vs baseline: 1.0399x; 1.0399x over previous
import functools

import jax
import jax.numpy as jnp
from jax import lax
from jax.experimental import pallas as pl
from jax.experimental.pallas import tpu as pltpu
from jax.experimental.pallas import tpu_sc as plsc

_BATCH = 16384
_DIM = 128
_NUM_CORES = 2
_NUM_SUBCORES = 16
_NW = _NUM_CORES * _NUM_SUBCORES
_B_SC = 14336
_CHUNK = _B_SC // _NW
_S = 112
_NSUB = _CHUNK // _S
_LANES = 16

_mesh = plsc.VectorSubcoreMesh(core_axis_name="c", subcore_axis_name="s")


@functools.partial(
    pl.kernel,
    out_type=jax.ShapeDtypeStruct((_NW, _LANES), jnp.float32),
    mesh=_mesh,
    scratch_types=[
        pltpu.VMEM((_CHUNK,), jnp.int32),
        pltpu.VMEM((3, _S, _DIM), jnp.float32),
        pltpu.VMEM((_LANES,), jnp.float32),
        [pltpu.SemaphoreType.DMA] * 3,
        [pltpu.SemaphoreType.DMA] * 3,
    ],
)
def _center_loss_partials(x_hbm, labels_hbm, neg_centers_hbm, out_hbm,
                          idx_v, xc_v, acc_v, sems_x, sems_a):
    wid = lax.axis_index("s") * _NUM_CORES + lax.axis_index("c")
    base = wid * _CHUNK
    pltpu.sync_copy(labels_hbm.at[pl.ds(base, _CHUNK)], idx_v)

    def start_x(h):
        b = h % 3
        return pltpu.async_copy(
            x_hbm.at[pl.ds(base + h * _S, _S)], xc_v.at[b], sems_x[b])

    def start_add(h):
        b = h % 3
        return pltpu.async_copy(
            neg_centers_hbm.at[idx_v.at[pl.ds(h * _S, _S)]], xc_v.at[b],
            sems_a[b], add=True)

    cp_x = [None, None, None]
    cp_a = [None, None, None]
    cp_x[0] = start_x(0)
    cp_x[1] = start_x(1)
    cp_x[0].wait()
    cp_a[0] = start_add(0)
    cp_x[2] = start_x(2)

    acc = jnp.zeros((_LANES,), jnp.float32)
    for h in range(_NSUB):
        b = h % 3
        cp_a[b].wait()
        if h + 1 < _NSUB:
            cp_x[(h + 1) % 3].wait()
            cp_a[(h + 1) % 3] = start_add(h + 1)

        def row_body(r, a):
            for j in range(_DIM // _LANES):
                d = xc_v[b, r, pl.ds(j * _LANES, _LANES)]
                a = a + d * d
            return a

        acc = lax.fori_loop(0, _S, row_body, acc)
        if h + 3 < _NSUB:
            cp_x[(h + 3) % 3] = start_x(h + 3)

    acc_v[...] = acc
    pltpu.sync_copy(acc_v, out_hbm.at[wid])


def kernel(x, labels, centers):
    partials = _center_loss_partials(x, labels, -centers)
    c_tc = jnp.take(centers, labels[_B_SC:], axis=0)
    tc_part = jnp.sum(jnp.square(x[_B_SC:] - c_tc))
    return (jnp.sum(partials) + tc_part) * (0.5 / _BATCH)
